# baseline (speedup 1.0000x reference)
import functools

import jax
import jax.numpy as jnp
from jax import lax
from jax.experimental import pallas as pl
from jax.experimental.pallas import tpu as pltpu
from jax.experimental.pallas import tpu_sc as plsc


def _make_gather(VP, D, B0, H):
    NC, NS = 2, 16
    NW = NC * NS
    BT = B0 // 128
    NR = H * D // 8 // NW
    NQ = 4
    CB = B0 // NQ
    RW = CB // 128 * 8
    n = NR * NQ
    assert n % 2 == 0
    mesh = plsc.VectorSubcoreMesh(core_axis_name="c", subcore_axis_name="s")

    @functools.partial(
        pl.kernel,
        mesh=mesh,
        compiler_params=pltpu.CompilerParams(needs_layout_passes=False),
        out_type=jax.ShapeDtypeStruct((H * D // 8, BT * 8, 128), jnp.float32),
        scratch_types=[
            pltpu.VMEM((VP, D), jnp.float32),
            pltpu.VMEM((CB // 128, 128), jnp.int32),
            pltpu.VMEM((CB // 128, 128), jnp.int32),
            pltpu.VMEM((RW, 128), jnp.float32),
            pltpu.VMEM((RW, 128), jnp.float32),
            pltpu.SemaphoreType.DMA,
            pltpu.SemaphoreType.DMA,
            pltpu.SemaphoreType.DMA,
            pltpu.SemaphoreType.DMA,
        ],
    )
    def k(tct_hbm, table_hbm, out_hbm, table_v, i0, i1, r0, r1, l0, l1, w0, w1):
        ibuf = (i0, i1)
        rbuf = (r0, r1)
        lsem = (l0, l1)
        wsem = (w0, w1)
        wid = lax.axis_index("s") * NC + lax.axis_index("c")
        row0 = wid * NR

        def coords(q):
            row = row0 + q // NQ
            return row, row // (D // 8), row % (D // 8), q % NQ

        def startL(q, b):
            row, h, dt, qq = coords(q)
            pltpu.async_copy(
                tct_hbm.at[h, pl.ds(qq * (CB // 128), CB // 128), :],
                ibuf[b],
                lsem[b],
            )

        def waitL(b):
            pltpu.make_async_copy(
                tct_hbm.at[0, pl.ds(0, CB // 128), :], ibuf[b], lsem[b]
            ).wait()

        def startW(q, b):
            row, h, dt, qq = coords(q)
            pltpu.async_copy(
                rbuf[b], out_hbm.at[row, pl.ds(qq * RW, RW), :], wsem[b]
            )

        def waitW(b):
            pltpu.make_async_copy(
                rbuf[b], out_hbm.at[0, pl.ds(0, RW), :], wsem[b]
            ).wait()

        def compute(q, b):
            row, h, dt, qq = coords(q)
            src = ibuf[b]
            dst = rbuf[b]
            vdt = jnp.zeros((16,), jnp.int32) + dt * 8
            dvs = [vdt + r for r in range(8)]

            @plsc.parallel_loop(0, CB // 16, step=1)
            def _(g):
                sv = src[g // 8, pl.ds((g % 8) * 16, 16)]
                bt8 = (g // 8) * 8
                c0 = (g % 8) * 16
                for r in range(8):
                    vals = plsc.load_gather(table_v, [sv, dvs[r]])
                    dst[bt8 + r, pl.ds(c0, 16)] = vals

        pltpu.sync_copy(table_hbm, table_v)
        startL(0, 0)
        startL(1, 1)

        @pl.loop(0, n, step=2)
        def _(t):
            for d2 in range(2):
                q = t + d2
                b = d2
                waitL(b)

                @pl.when(q >= 2)
                def _():
                    waitW(b)

                compute(q, b)
                startW(q, b)

                @pl.when(q + 2 < n)
                def _():
                    startL(q + 2, b)

        waitW(0)
        waitW(1)

    return k


def kernel(tc, embedding):
    B0, H = tc.shape
    V, D = embedding.shape
    VP = V + (-V) % 8
    BT = B0 // 128
    tct = tc.T.reshape(H, B0 // 128, 128)
    table = jnp.pad(embedding.astype(jnp.float32), ((0, VP - V), (0, 0)))
    x5 = _make_gather(VP, D, B0, H)(tct, table)
    x5 = x5.reshape(H, D // 8, BT, 8, 128)
    return x5.transpose(2, 4, 0, 1, 3).reshape(B0, H, D)


# speedup vs baseline: 6.5507x; 6.5507x over previous
import functools

import jax
import jax.numpy as jnp
from jax import lax
from jax.experimental import pallas as pl
from jax.experimental.pallas import tpu as pltpu
from jax.experimental.pallas import tpu_sc as plsc


def _make_gather(VP, D, B0, H):
    NC, NS = 2, 16
    NW = NC * NS
    BT = B0 // 128
    NR = H * D // 8 // NW
    NQ = 4
    CB = B0 // NQ
    RW = CB // 128 * 8
    n = NR * NQ
    assert n % 2 == 0
    mesh = plsc.VectorSubcoreMesh(core_axis_name="c", subcore_axis_name="s")

    @functools.partial(
        pl.kernel,
        mesh=mesh,
        compiler_params=pltpu.CompilerParams(needs_layout_passes=False),
        out_type=jax.ShapeDtypeStruct((H * D // 8, BT * 8, 128), jnp.float32),
        scratch_types=[
            pltpu.VMEM((D, VP), jnp.float32),
            pltpu.VMEM((CB // 128, 128), jnp.int32),
            pltpu.VMEM((CB // 128, 128), jnp.int32),
            pltpu.VMEM((RW, 128), jnp.float32),
            pltpu.VMEM((RW, 128), jnp.float32),
            pltpu.SemaphoreType.DMA,
            pltpu.SemaphoreType.DMA,
            pltpu.SemaphoreType.DMA,
            pltpu.SemaphoreType.DMA,
        ],
    )
    def k(tct_hbm, table_hbm, out_hbm, table_v, i0, i1, r0, r1, l0, l1, w0, w1):
        ibuf = (i0, i1)
        rbuf = (r0, r1)
        lsem = (l0, l1)
        wsem = (w0, w1)
        wid = lax.axis_index("s") * NC + lax.axis_index("c")
        row0 = wid * NR

        def coords(q):
            row = row0 + q // NQ
            return row, row // (D // 8), row % (D // 8), q % NQ

        def startL(q, b):
            row, h, dt, qq = coords(q)
            pltpu.async_copy(
                tct_hbm.at[h, pl.ds(qq * (CB // 128), CB // 128), :],
                ibuf[b],
                lsem[b],
            )

        def waitL(b):
            pltpu.make_async_copy(
                tct_hbm.at[0, pl.ds(0, CB // 128), :], ibuf[b], lsem[b]
            ).wait()

        def startW(q, b):
            row, h, dt, qq = coords(q)
            pltpu.async_copy(
                rbuf[b], out_hbm.at[row, pl.ds(qq * RW, RW), :], wsem[b]
            )

        def waitW(b):
            pltpu.make_async_copy(
                rbuf[b], out_hbm.at[0, pl.ds(0, RW), :], wsem[b]
            ).wait()

        def compute(q, b):
            row, h, dt, qq = coords(q)
            src = ibuf[b]
            dst = rbuf[b]
            vdt = jnp.zeros((16,), jnp.int32) + dt * 8
            dvs = [vdt + r for r in range(8)]

            @plsc.parallel_loop(0, CB // 16, step=1)
            def _(g):
                sv = src[g // 8, pl.ds((g % 8) * 16, 16)]
                bt8 = (g // 8) * 8
                c0 = (g % 8) * 16
                for r in range(8):
                    vals = plsc.load_gather(table_v, [dvs[r], sv])
                    dst[bt8 + r, pl.ds(c0, 16)] = vals

        pltpu.sync_copy(table_hbm, table_v)
        startL(0, 0)
        startL(1, 1)

        @pl.loop(0, n, step=2)
        def _(t):
            for d2 in range(2):
                q = t + d2
                b = d2
                waitL(b)

                @pl.when(q >= 2)
                def _():
                    waitW(b)

                compute(q, b)
                startW(q, b)

                @pl.when(q + 2 < n)
                def _():
                    startL(q + 2, b)

        waitW(0)
        waitW(1)

    return k


def kernel(tc, embedding):
    B0, H = tc.shape
    V, D = embedding.shape
    VP = V + (-V) % 8
    BT = B0 // 128
    tct = tc.T.reshape(H, B0 // 128, 128)
    table = jnp.pad(embedding.astype(jnp.float32).T, ((0, 0), (0, VP - V)))
    x5 = _make_gather(VP, D, B0, H)(tct, table)
    x5 = x5.reshape(H, D // 8, BT, 8, 128)
    return x5.transpose(2, 4, 0, 1, 3).reshape(B0, H, D)
